# Initial kernel scaffold; baseline (speedup 1.0000x reference)
import jax
import jax.numpy as jnp
from jax.experimental import pallas as pl


def kernel(x, ln_scale, ln_bias, router_W, shared_gate_up_W, shared_down_W, expert_gate_up_W, expert_down_W):
    raise NotImplementedError("write your pallas kernel here")


# speedup vs baseline: 1.4075x; 1.4075x over previous
import functools

import jax
import jax.numpy as jnp
from jax.experimental import pallas as pl
from jax.experimental.pallas import tpu as pltpu

D_MODEL = 768
NUM_EXPERTS = 8
ROUTED_DFF = 2304
SHARED_DFF = 768
CHUNK = 768
N_CHUNKS = ROUTED_DFF // CHUNK
SEQ = 2048


def _moe_kernel(x_ref, ln_scale_ref, ln_bias_ref, router_W_ref,
                gate_W_ref, up_W_ref, down_W_ref,
                sh_gate_ref, sh_up_ref, sh_down_ref,
                out_ref,
                xb_ref, i1_ref, i2_ref, w1_ref, w2_ref):
    e = pl.program_id(0)
    c = pl.program_id(1)

    @pl.when(jnp.logical_and(e == 0, c == 0))
    def _router():
        x = x_ref[...]
        mu = jnp.mean(x, axis=1, keepdims=True)
        xc = x - mu
        var = jnp.mean(xc * xc, axis=1, keepdims=True)
        xn = xc * jax.lax.rsqrt(var + 1e-5)
        xn = xn * ln_scale_ref[...] + ln_bias_ref[...]
        xb_ref[...] = xn.astype(jnp.bfloat16)
        logits = jax.lax.dot_general(
            xn.astype(jnp.bfloat16),
            router_W_ref[...].astype(jnp.bfloat16),
            (((1,), (1,)), ((), ())),
            preferred_element_type=jnp.float32)
        m = jnp.max(logits, axis=1, keepdims=True)
        ex = jnp.exp(logits - m)
        probs = ex / jnp.sum(ex, axis=1, keepdims=True)
        iota = jax.lax.broadcasted_iota(jnp.int32, probs.shape, 1)
        p1 = jnp.max(probs, axis=1, keepdims=True)
        i1 = jnp.min(jnp.where(probs == p1, iota, NUM_EXPERTS), axis=1,
                     keepdims=True)
        masked = jnp.where(iota == i1, -1.0, probs)
        p2 = jnp.max(masked, axis=1, keepdims=True)
        i2 = jnp.min(jnp.where(masked == p2, iota, NUM_EXPERTS), axis=1,
                     keepdims=True)
        a = jnp.exp(p1 - p1)
        b = jnp.exp(p2 - p1)
        denom = a + b
        i1_ref[...] = i1.astype(jnp.float32)
        i2_ref[...] = i2.astype(jnp.float32)
        w1_ref[...] = a / denom
        w2_ref[...] = b / denom
        out_ref[...] = jnp.zeros_like(out_ref)

    xb = xb_ref[...]
    g = jax.lax.dot_general(xb, gate_W_ref[0], (((1,), (1,)), ((), ())),
                            preferred_element_type=jnp.float32)
    u = jax.lax.dot_general(xb, up_W_ref[0], (((1,), (1,)), ((), ())),
                            preferred_element_type=jnp.float32)
    h = (g * jax.nn.sigmoid(g)) * u
    y = jax.lax.dot_general(h.astype(jnp.bfloat16), down_W_ref[0],
                            (((1,), (1,)), ((), ())),
                            preferred_element_type=jnp.float32)
    ef = jnp.float32(0) + e
    ge = (jnp.where(i1_ref[...] == ef, w1_ref[...], 0.0)
          + jnp.where(i2_ref[...] == ef, w2_ref[...], 0.0))
    out_ref[...] += ge * y

    @pl.when(jnp.logical_and(e == NUM_EXPERTS - 1, c == N_CHUNKS - 1))
    def _shared_and_out():
        xb2 = xb_ref[...]
        sg = jax.lax.dot_general(xb2, sh_gate_ref[...],
                                 (((1,), (1,)), ((), ())),
                                 preferred_element_type=jnp.float32)
        su = jax.lax.dot_general(xb2, sh_up_ref[...],
                                 (((1,), (1,)), ((), ())),
                                 preferred_element_type=jnp.float32)
        sh = (sg * jax.nn.sigmoid(sg)) * su
        ys = jax.lax.dot_general(sh.astype(jnp.bfloat16), sh_down_ref[...],
                                 (((1,), (1,)), ((), ())),
                                 preferred_element_type=jnp.float32)
        out_ref[...] += ys


@jax.jit
def kernel(x, ln_scale, ln_bias, router_W, shared_gate_up_W, shared_down_W,
           expert_gate_up_W, expert_down_W):
    B, S, D = x.shape
    x2 = x.reshape(S, D)
    gate_W = expert_gate_up_W[:, :ROUTED_DFF, :].astype(jnp.bfloat16)
    up_W = expert_gate_up_W[:, ROUTED_DFF:, :].astype(jnp.bfloat16)
    down_W = expert_down_W.astype(jnp.bfloat16)
    sh_gate = shared_gate_up_W[:SHARED_DFF, :].astype(jnp.bfloat16)
    sh_up = shared_gate_up_W[SHARED_DFF:, :].astype(jnp.bfloat16)
    sh_down = shared_down_W.astype(jnp.bfloat16)
    ln_scale2 = ln_scale.reshape(1, D)
    ln_bias2 = ln_bias.reshape(1, D)

    grid = (NUM_EXPERTS, N_CHUNKS)
    out = pl.pallas_call(
        _moe_kernel,
        grid=grid,
        in_specs=[
            pl.BlockSpec((S, D), lambda e, c: (0, 0)),
            pl.BlockSpec((1, D), lambda e, c: (0, 0)),
            pl.BlockSpec((1, D), lambda e, c: (0, 0)),
            pl.BlockSpec((NUM_EXPERTS, D), lambda e, c: (0, 0)),
            pl.BlockSpec((1, CHUNK, D), lambda e, c: (e, c, 0)),
            pl.BlockSpec((1, CHUNK, D), lambda e, c: (e, c, 0)),
            pl.BlockSpec((1, D, CHUNK), lambda e, c: (e, 0, c)),
            pl.BlockSpec((SHARED_DFF, D), lambda e, c: (0, 0)),
            pl.BlockSpec((SHARED_DFF, D), lambda e, c: (0, 0)),
            pl.BlockSpec((D, SHARED_DFF), lambda e, c: (0, 0)),
        ],
        out_specs=pl.BlockSpec((S, D), lambda e, c: (0, 0)),
        out_shape=jax.ShapeDtypeStruct((S, D), jnp.float32),
        scratch_shapes=[
            pltpu.VMEM((S, D), jnp.bfloat16),
            pltpu.VMEM((S, 1), jnp.float32),
            pltpu.VMEM((S, 1), jnp.float32),
            pltpu.VMEM((S, 1), jnp.float32),
            pltpu.VMEM((S, 1), jnp.float32),
        ],
        compiler_params=pltpu.CompilerParams(
            dimension_semantics=("arbitrary", "arbitrary"),
        ),
    )(x2, ln_scale2, ln_bias2, router_W, gate_W, up_W, down_W,
      sh_gate, sh_up, sh_down)
    return out.reshape(B, S, D)
